# baseline (speedup 1.0000x reference)
import functools

import jax
import jax.numpy as jnp
from jax import lax
from jax.experimental import pallas as pl
from jax.experimental.pallas import tpu as pltpu
from jax.experimental.pallas import tpu_sc as plsc

B = 4
S = 2048
H = 128
L = 16
NVPR = H // L
NC = 2
NS = 16
NW = NC * NS
TOK = B * S
TPW = TOK // NW
EPS = 1e-5


def _body(idx_hbm, tt_hbm, word_hbm, pos_hbm, type_hbm, gb_hbm, out_hbm,
          idx_v, tt_v, rows_v, pos_v, type_v, gb_v, sem):
    c = lax.axis_index("c")
    s = lax.axis_index("s")
    wid = s * NC + c
    base = wid * TPW
    pos_base = lax.rem(base, S)

    pltpu.sync_copy(idx_hbm.at[pl.ds(base, TPW)], idx_v)
    pltpu.sync_copy(tt_hbm.at[pl.ds(base, TPW)], tt_v)

    cp_w = pltpu.async_copy(word_hbm.at[idx_v], rows_v, sem)
    cp_t = pltpu.async_copy(type_hbm.at[tt_v], type_v, sem)
    pltpu.sync_copy(pos_hbm.at[pl.ds(pos_base, TPW)], pos_v)
    pltpu.sync_copy(gb_hbm, gb_v)
    cp_w.wait()
    cp_t.wait()

    gamma = [gb_v[0, pl.ds(j * L, L)] for j in range(NVPR)]
    beta = [gb_v[1, pl.ds(j * L, L)] for j in range(NVPR)]
    inv_h = 1.0 / H

    def token_body(i, carry):
        v = []
        for j in range(NVPR):
            sl = pl.ds(j * L, L)
            v.append(rows_v[i, sl] + pos_v[i, sl] + type_v[i, sl])
        su = v[0]
        sq = v[0] * v[0]
        for j in range(1, NVPR):
            su = su + v[j]
            sq = sq + v[j] * v[j]
        tot = jnp.sum(su)
        tot2 = jnp.sum(sq)
        mean = tot * inv_h
        var = tot2 * inv_h - mean * mean
        x = jnp.broadcast_to(var + EPS, (L,))
        bits = lax.bitcast_convert_type(x, jnp.int32)
        y = lax.bitcast_convert_type(0x5F3759DF - (bits >> 1), jnp.float32)
        for _ in range(3):
            y = y * (1.5 - 0.5 * x * y * y)
        m = jnp.broadcast_to(mean, (L,))
        for j in range(NVPR):
            rows_v[i, pl.ds(j * L, L)] = (v[j] - m) * y * gamma[j] + beta[j]
        return carry

    lax.fori_loop(0, TPW, token_body, 0)
    pltpu.sync_copy(rows_v, out_hbm.at[pl.ds(base, TPW)])


@functools.partial(jax.jit, static_argnames=())
def _run(idx, tt, word_table, pos_table, type_table, gb):
    mesh = plsc.VectorSubcoreMesh(core_axis_name="c", subcore_axis_name="s")
    fn = functools.partial(
        pl.kernel,
        mesh=mesh,
        out_type=jax.ShapeDtypeStruct((TOK, H), jnp.float32),
        scratch_types=[
            pltpu.VMEM((TPW,), jnp.int32),
            pltpu.VMEM((TPW,), jnp.int32),
            pltpu.VMEM((TPW, H), jnp.float32),
            pltpu.VMEM((TPW, H), jnp.float32),
            pltpu.VMEM((TPW, H), jnp.float32),
            pltpu.VMEM((2, H), jnp.float32),
            pltpu.SemaphoreType.DMA,
        ],
        compiler_params=pltpu.CompilerParams(needs_layout_passes=False),
    )(_body)
    return fn(idx, tt, word_table, pos_table, type_table, gb)


def kernel(indices, token_type, word_table, pos_table, type_table, ln_gamma, ln_beta):
    idx = indices.reshape(-1).astype(jnp.int32)
    tt = token_type.reshape(-1).astype(jnp.int32)
    gb = jnp.stack([ln_gamma, ln_beta]).astype(jnp.float32)
    out = _run(idx, tt, word_table, pos_table, type_table, gb)
    return out.reshape(indices.shape + (H,))
